# baseline (speedup 1.0000x reference)
import functools

import jax
import jax.numpy as jnp
from jax import lax
from jax.experimental import pallas as pl
from jax.experimental.pallas import tpu as pltpu
from jax.experimental.pallas import tpu_sc as plsc

N = 10000
E = 160000
D = 256
H = 128
SLOPE = 0.01

NS = 16
K = 120
NBUF = 3
CPT = 84
EPT = K * CPT
EPAD = NS * EPT
NACC = N + 16
NIT2 = CPT // (2 * NBUF)
K1 = 64
RPT = 624
REM0 = NS * RPT
REM = N - REM0

ROW_BLK = 1000


def _lrelu(v):
    return jnp.where(v >= 0, v, SLOPE * v)


def _head_body(pref_ref, feat_ref, mw_ref, mb_ref, cw_ref, xn_ref, h_ref):
    i = pl.program_id(0)
    temp = (
        jnp.dot(feat_ref[...], mw_ref[...], preferred_element_type=jnp.float32)
        + mb_ref[...]
    )
    x = jnp.where(i < 2, pref_ref[...], temp)
    n = jnp.sqrt(jnp.sum(x * x, axis=1, keepdims=True))
    xn = x / jnp.maximum(n, 1e-12)
    xn_ref[...] = xn
    h_ref[...] = jnp.dot(xn, cw_ref[...], preferred_element_type=jnp.float32)


def _head(preference, features, mlp_w, mlp_b, conv_w):
    kdim = features.shape[1]
    return pl.pallas_call(
        _head_body,
        grid=(N // ROW_BLK,),
        in_specs=[
            pl.BlockSpec((ROW_BLK, D), lambda i: (jnp.minimum(i, 1), 0)),
            pl.BlockSpec((ROW_BLK, kdim),
                         lambda i: (jnp.maximum(i - 2, 0), 0)),
            pl.BlockSpec((kdim, D), lambda i: (0, 0)),
            pl.BlockSpec((1, D), lambda i: (0, 0)),
            pl.BlockSpec((D, D), lambda i: (0, 0)),
        ],
        out_specs=[
            pl.BlockSpec((ROW_BLK, D), lambda i: (i, 0)),
            pl.BlockSpec((ROW_BLK, D), lambda i: (i, 0)),
        ],
        out_shape=[
            jax.ShapeDtypeStruct((N, D), jnp.float32),
            jax.ShapeDtypeStruct((N, D), jnp.float32),
        ],
    )(preference, features, mlp_w, mlp_b.reshape(1, D), conv_w)


def _layer_body(a0_ref, a1_ref, x_ref, id_ref, lw_ref, lb_ref, gw_ref,
                gb_ref, cw_ref, xo_ref, ho_ref):
    h0 = _lrelu(a0_ref[0])
    h1 = _lrelu(a1_ref[0])
    xh = _lrelu(
        jnp.dot(x_ref[...], lw_ref[...], preferred_element_type=jnp.float32)
        + lb_ref[...]
    ) + id_ref[...]
    xo = _lrelu(
        jnp.dot(h0, gw_ref[0:H, :], preferred_element_type=jnp.float32)
        + jnp.dot(h1, gw_ref[H:D, :], preferred_element_type=jnp.float32)
        + jnp.dot(xh, gw_ref[D:2 * D, :], preferred_element_type=jnp.float32)
        + gb_ref[...]
    )
    xo_ref[...] = xo
    if ho_ref is not None:
        ho_ref[...] = jnp.dot(
            xo, cw_ref[...], preferred_element_type=jnp.float32
        )


def _layer(agg, x, id_emb, lin_w, lin_b, g_w, g_b, conv_w=None):
    has_conv = conv_w is not None
    if has_conv:
        body = _layer_body
    else:
        body = lambda a, a2, b, c, d, e, f, g, xo: _layer_body(
            a, a2, b, c, d, e, f, g, None, xo, None)
    in_specs = [
        pl.BlockSpec((1, ROW_BLK, H), lambda i: (0, i, 0)),
        pl.BlockSpec((1, ROW_BLK, H), lambda i: (1, i, 0)),
        pl.BlockSpec((ROW_BLK, D), lambda i: (i, 0)),
        pl.BlockSpec((ROW_BLK, D), lambda i: (i, 0)),
        pl.BlockSpec((D, D), lambda i: (0, 0)),
        pl.BlockSpec((1, D), lambda i: (0, 0)),
        pl.BlockSpec((2 * D, D), lambda i: (0, 0)),
        pl.BlockSpec((1, D), lambda i: (0, 0)),
    ]
    args = [agg, agg, x, id_emb, lin_w, lin_b.reshape(1, D), g_w,
            g_b.reshape(1, D)]
    out_specs = [pl.BlockSpec((ROW_BLK, D), lambda i: (i, 0))]
    out_shape = [jax.ShapeDtypeStruct((N, D), jnp.float32)]
    if has_conv:
        in_specs.append(pl.BlockSpec((D, D), lambda i: (0, 0)))
        args.append(conv_w)
        out_specs.append(pl.BlockSpec((ROW_BLK, D), lambda i: (i, 0)))
        out_shape.append(jax.ShapeDtypeStruct((N, D), jnp.float32))
        res = pl.pallas_call(
            body, grid=(N // ROW_BLK,), in_specs=in_specs,
            out_specs=out_specs, out_shape=out_shape)(*args)
        return res[0], res[1]
    res = pl.pallas_call(
        body, grid=(N // ROW_BLK,), in_specs=in_specs,
        out_specs=out_specs[0], out_shape=out_shape[0])(*args)
    return res, None


@functools.cache
def _make_sc_aggregate():
    mesh = plsc.VectorSubcoreMesh(core_axis_name="c", subcore_axis_name="s")

    @functools.partial(
        pl.kernel,
        out_type=jax.ShapeDtypeStruct((2, N, H), jnp.float32),
        mesh=mesh,
        scratch_types=[
            [[pltpu.VMEM((2, K), jnp.int32) for _ in range(NBUF)]
             for _ in range(2)],
            [pltpu.VMEM((K, H), jnp.float32) for _ in range(NBUF)],
            [pltpu.SemaphoreType.DMA for _ in range(NBUF)],
            [pltpu.SemaphoreType.DMA for _ in range(NBUF)],
            [pltpu.SemaphoreType.DMA for _ in range(NBUF)],
            pltpu.VMEM_SHARED((NACC, H), jnp.float32),
        ],
    )
    def _sc_aggregate(table, comb, zrows, out,
                      idx_v, rows, isem, gsem, ssem, acc):
        c = lax.axis_index("c")
        s = lax.axis_index("s")
        row0 = s * RPT
        cbase = (c * NS + s) * CPT

        pltpu.sync_copy(zrows, acc.at[pl.ds(row0, RPT)])

        @pl.when(s == 0)
        def _():
            pltpu.sync_copy(zrows.at[pl.ds(0, REM)], acc.at[pl.ds(REM0, REM)])

        plsc.subcore_barrier()

        def idx_load(j, b, bank):
            return (
                pltpu.make_async_copy(
                    comb.at[cbase + j], idx_v[bank][b], isem[b]),
            )

        def gather(b, bank):
            return (
                pltpu.make_async_copy(
                    table.at[idx_v[bank][b].at[0, pl.ds(0, K1)]],
                    rows[b].at[pl.ds(0, K1)], gsem[b]),
                pltpu.make_async_copy(
                    table.at[idx_v[bank][b].at[0, pl.ds(K1, K - K1)]],
                    rows[b].at[pl.ds(K1, K - K1)], gsem[b]),
            )

        def scatter(b, bank):
            return pltpu.make_async_copy(
                rows[b], acc.at[idx_v[bank][b].at[1]], ssem[b])

        for b in range(NBUF):
            for d in idx_load(b, b, 0):
                d.start()

        def group(i, g):
            j0 = (2 * i + g) * NBUF
            for b in range(NBUF):
                if g == 1:
                    scatter(b, 0).wait()
                else:
                    @pl.when(i > 0)
                    def _():
                        scatter(b, 1).wait()
                for d in idx_load(j0 + b, b, g):
                    d.wait()
                for d in gather(b, g):
                    d.start()
            for b in range(NBUF):
                for d in gather(b, g):
                    d.wait()
                if g == 0:
                    for d in idx_load(j0 + NBUF + b, b, 1):
                        d.start()
                else:
                    @pl.when(i < NIT2 - 1)
                    def _():
                        for d in idx_load(j0 + NBUF + b, b, 0):
                            d.start()
                pltpu.async_copy(rows[b], acc.at[idx_v[g][b].at[1]], ssem[b],
                                 add=True)

        def body(i, carry):
            group(i, 0)
            group(i, 1)
            return carry

        lax.fori_loop(0, NIT2, body, 0)
        for b in range(NBUF):
            scatter(b, 1).wait()

        plsc.subcore_barrier()
        pltpu.sync_copy(acc.at[pl.ds(row0, RPT)],
                        out.at[c, pl.ds(row0, RPT)])

        @pl.when(s == 0)
        def _():
            pltpu.sync_copy(acc.at[pl.ds(REM0, REM)],
                            out.at[c, pl.ds(REM0, REM)])

    return _sc_aggregate


def _aggregate(h, comb, zrows):
    return _make_sc_aggregate()(h.reshape(2 * N, H), comb, zrows)


def kernel(features, id_embedding, edge_index, preference, mlp_w, mlp_b,
           conv1_w, lin1_w, lin1_b, g1_w, g1_b, conv2_w, lin2_w, lin2_b,
           g2_w, g2_b, conv3_w, lin3_w, lin3_b, g3_w, g3_b):
    src = edge_index[0]
    dst = edge_index[1]
    npad = EPAD - E
    src_p = jnp.concatenate([src, jnp.zeros((npad,), jnp.int32)])
    dst_p = jnp.concatenate([dst, jnp.full((npad,), N, jnp.int32)])
    src2 = jnp.stack([2 * src_p, 2 * src_p + 1]).reshape(2, NS * CPT, K)
    dstc = jnp.broadcast_to(dst_p.reshape(1, NS * CPT, K), (2, NS * CPT, K))
    comb = jnp.stack([src2, dstc], axis=2).reshape(2 * NS * CPT, 2, K)
    zrows = jnp.zeros((RPT, H), jnp.float32)

    x, h = _head(preference, features, mlp_w, mlp_b, conv1_w)

    agg = _aggregate(h, comb, zrows)
    x, h = _layer(agg, x, id_embedding, lin1_w, lin1_b, g1_w, g1_b, conv2_w)

    agg = _aggregate(h, comb, zrows)
    x, h = _layer(agg, x, id_embedding, lin2_w, lin2_b, g2_w, g2_b, conv3_w)

    agg = _aggregate(h, comb, zrows)
    x, _ = _layer(agg, x, id_embedding, lin3_w, lin3_b, g3_w, g3_b)
    return x


# speedup vs baseline: 1.0214x; 1.0214x over previous
import functools

import jax
import jax.numpy as jnp
from jax import lax
from jax.experimental import pallas as pl
from jax.experimental.pallas import tpu as pltpu
from jax.experimental.pallas import tpu_sc as plsc

N = 10000
E = 160000
D = 256
H = 128
SLOPE = 0.01

NS = 16
K = 120
NBUF = 3
CPT = 84
EPT = K * CPT
EPAD = NS * EPT
NACC = N + 16
NIT2 = CPT // (2 * NBUF)
K1 = 64
RPT = 624
REM0 = NS * RPT
REM = N - REM0

ROW_BLK = 1000


def _lrelu(v):
    return jnp.where(v >= 0, v, SLOPE * v)


def _head_body(pref_ref, feat_ref, mw_ref, mb_ref, cw_ref, xn_ref, h_ref):
    i = pl.program_id(0)
    temp = (
        jnp.dot(feat_ref[...], mw_ref[...], preferred_element_type=jnp.float32)
        + mb_ref[...]
    )
    x = jnp.where(i < 2, pref_ref[...], temp)
    n = jnp.sqrt(jnp.sum(x * x, axis=1, keepdims=True))
    xn = x / jnp.maximum(n, 1e-12)
    xn_ref[...] = xn
    h_ref[...] = jnp.dot(xn, cw_ref[...], preferred_element_type=jnp.float32)


def _head(preference, features, mlp_w, mlp_b, conv_w):
    kdim = features.shape[1]
    return pl.pallas_call(
        _head_body,
        grid=(N // ROW_BLK,),
        in_specs=[
            pl.BlockSpec((ROW_BLK, D), lambda i: (jnp.minimum(i, 1), 0)),
            pl.BlockSpec((ROW_BLK, kdim),
                         lambda i: (jnp.maximum(i - 2, 0), 0)),
            pl.BlockSpec((kdim, D), lambda i: (0, 0)),
            pl.BlockSpec((1, D), lambda i: (0, 0)),
            pl.BlockSpec((D, D), lambda i: (0, 0)),
        ],
        out_specs=[
            pl.BlockSpec((ROW_BLK, D), lambda i: (i, 0)),
            pl.BlockSpec((ROW_BLK, D), lambda i: (i, 0)),
        ],
        out_shape=[
            jax.ShapeDtypeStruct((N, D), jnp.float32),
            jax.ShapeDtypeStruct((N, D), jnp.float32),
        ],
    )(preference, features, mlp_w, mlp_b.reshape(1, D), conv_w)


def _layer_body(a0_ref, a1_ref, x_ref, id_ref, lw_ref, lb_ref, gw_ref,
                gb_ref, cw_ref, xo_ref, ho_ref):
    h0 = _lrelu(a0_ref[0])
    h1 = _lrelu(a1_ref[0])
    xh = _lrelu(
        jnp.dot(x_ref[...], lw_ref[...], preferred_element_type=jnp.float32)
        + lb_ref[...]
    ) + id_ref[...]
    xo = _lrelu(
        jnp.dot(h0, gw_ref[0:H, :], preferred_element_type=jnp.float32)
        + jnp.dot(h1, gw_ref[H:D, :], preferred_element_type=jnp.float32)
        + jnp.dot(xh, gw_ref[D:2 * D, :], preferred_element_type=jnp.float32)
        + gb_ref[...]
    )
    xo_ref[...] = xo
    if ho_ref is not None:
        ho_ref[...] = jnp.dot(
            xo, cw_ref[...], preferred_element_type=jnp.float32
        )


def _layer(agg, x, id_emb, lin_w, lin_b, g_w, g_b, conv_w=None):
    has_conv = conv_w is not None
    if has_conv:
        body = _layer_body
    else:
        body = lambda a, a2, b, c, d, e, f, g, xo: _layer_body(
            a, a2, b, c, d, e, f, g, None, xo, None)
    in_specs = [
        pl.BlockSpec((1, ROW_BLK, H), lambda i: (0, i, 0)),
        pl.BlockSpec((1, ROW_BLK, H), lambda i: (1, i, 0)),
        pl.BlockSpec((ROW_BLK, D), lambda i: (i, 0)),
        pl.BlockSpec((ROW_BLK, D), lambda i: (i, 0)),
        pl.BlockSpec((D, D), lambda i: (0, 0)),
        pl.BlockSpec((1, D), lambda i: (0, 0)),
        pl.BlockSpec((2 * D, D), lambda i: (0, 0)),
        pl.BlockSpec((1, D), lambda i: (0, 0)),
    ]
    args = [agg, agg, x, id_emb, lin_w, lin_b.reshape(1, D), g_w,
            g_b.reshape(1, D)]
    out_specs = [pl.BlockSpec((ROW_BLK, D), lambda i: (i, 0))]
    out_shape = [jax.ShapeDtypeStruct((N, D), jnp.float32)]
    if has_conv:
        in_specs.append(pl.BlockSpec((D, D), lambda i: (0, 0)))
        args.append(conv_w)
        out_specs.append(pl.BlockSpec((ROW_BLK, D), lambda i: (i, 0)))
        out_shape.append(jax.ShapeDtypeStruct((N, D), jnp.float32))
        res = pl.pallas_call(
            body, grid=(N // ROW_BLK,), in_specs=in_specs,
            out_specs=out_specs, out_shape=out_shape)(*args)
        return res[0], res[1]
    res = pl.pallas_call(
        body, grid=(N // ROW_BLK,), in_specs=in_specs,
        out_specs=out_specs[0], out_shape=out_shape[0])(*args)
    return res, None


@functools.cache
def _make_sc_aggregate():
    mesh = plsc.VectorSubcoreMesh(core_axis_name="c", subcore_axis_name="s")

    @functools.partial(
        pl.kernel,
        out_type=jax.ShapeDtypeStruct((2, N, H), jnp.float32),
        mesh=mesh,
        scratch_types=[
            [pltpu.VMEM((K,), jnp.int32) for _ in range(NBUF)],
            [[pltpu.VMEM((K,), jnp.int32) for _ in range(NBUF)]
             for _ in range(2)],
            [pltpu.VMEM((K, H), jnp.float32) for _ in range(NBUF)],
            [pltpu.SemaphoreType.DMA for _ in range(NBUF)],
            [pltpu.SemaphoreType.DMA for _ in range(NBUF)],
            [pltpu.SemaphoreType.DMA for _ in range(NBUF)],
            pltpu.VMEM_SHARED((NACC, H), jnp.float32),
        ],
    )
    def _sc_aggregate(table, src2, dst, zrows, out,
                      src_v, dst_v, rows, isem, gsem, ssem, acc):
        c = lax.axis_index("c")
        s = lax.axis_index("s")
        row0 = s * RPT
        sbase = c * EPAD + s * EPT
        dbase = s * EPT

        pltpu.sync_copy(zrows, acc.at[pl.ds(row0, RPT)])

        @pl.when(s == 0)
        def _():
            pltpu.sync_copy(zrows.at[pl.ds(0, REM)], acc.at[pl.ds(REM0, REM)])

        plsc.subcore_barrier()

        def idx_load(j, b, bank):
            return (
                pltpu.make_async_copy(
                    src2.at[pl.ds(sbase + j * K, K)], src_v[b], isem[b]),
                pltpu.make_async_copy(
                    dst.at[pl.ds(dbase + j * K, K)], dst_v[bank][b], isem[b]),
            )

        def gather(b):
            return (
                pltpu.make_async_copy(
                    table.at[src_v[b].at[pl.ds(0, K1)]],
                    rows[b].at[pl.ds(0, K1)], gsem[b]),
                pltpu.make_async_copy(
                    table.at[src_v[b].at[pl.ds(K1, K - K1)]],
                    rows[b].at[pl.ds(K1, K - K1)], gsem[b]),
            )

        def scatter(b, bank):
            return pltpu.make_async_copy(
                rows[b], acc.at[dst_v[bank][b]], ssem[b])

        for b in range(NBUF):
            for d in idx_load(b, b, 0):
                d.start()

        def group(i, g):
            j0 = (2 * i + g) * NBUF
            for b in range(NBUF):
                if g == 1:
                    scatter(b, 0).wait()
                else:
                    @pl.when(i > 0)
                    def _():
                        scatter(b, 1).wait()
                for d in idx_load(j0 + b, b, g):
                    d.wait()
                for d in gather(b):
                    d.start()
            for b in range(NBUF):
                for d in gather(b):
                    d.wait()
                if g == 0:
                    for d in idx_load(j0 + NBUF + b, b, 1):
                        d.start()
                else:
                    @pl.when(i < NIT2 - 1)
                    def _():
                        for d in idx_load(j0 + NBUF + b, b, 0):
                            d.start()
                pltpu.async_copy(rows[b], acc.at[dst_v[g][b]], ssem[b],
                                 add=True)

        def body(i, carry):
            group(i, 0)
            group(i, 1)
            return carry

        lax.fori_loop(0, NIT2, body, 0)
        for b in range(NBUF):
            scatter(b, 1).wait()

        plsc.subcore_barrier()
        pltpu.sync_copy(acc.at[pl.ds(row0, RPT)],
                        out.at[c, pl.ds(row0, RPT)])

        @pl.when(s == 0)
        def _():
            pltpu.sync_copy(acc.at[pl.ds(REM0, REM)],
                            out.at[c, pl.ds(REM0, REM)])

    return _sc_aggregate


def _aggregate(h, src2, dst, zrows):
    return _make_sc_aggregate()(h.reshape(2 * N, H), src2, dst, zrows)


def kernel(features, id_embedding, edge_index, preference, mlp_w, mlp_b,
           conv1_w, lin1_w, lin1_b, g1_w, g1_b, conv2_w, lin2_w, lin2_b,
           g2_w, g2_b, conv3_w, lin3_w, lin3_b, g3_w, g3_b):
    src = edge_index[0]
    dst = edge_index[1]
    npad = EPAD - E
    src_p = jnp.concatenate([src, jnp.zeros((npad,), jnp.int32)])
    dst_p = jnp.concatenate([dst, jnp.full((npad,), N, jnp.int32)])
    src2 = jnp.concatenate([2 * src_p, 2 * src_p + 1])
    dst2 = dst_p
    zrows = jnp.zeros((RPT, H), jnp.float32)

    x, h = _head(preference, features, mlp_w, mlp_b, conv1_w)

    agg = _aggregate(h, src2, dst2, zrows)
    x, h = _layer(agg, x, id_embedding, lin1_w, lin1_b, g1_w, g1_b, conv2_w)

    agg = _aggregate(h, src2, dst2, zrows)
    x, h = _layer(agg, x, id_embedding, lin2_w, lin2_b, g2_w, g2_b, conv3_w)

    agg = _aggregate(h, src2, dst2, zrows)
    x, _ = _layer(agg, x, id_embedding, lin3_w, lin3_b, g3_w, g3_b)
    return x
